# baseline (speedup 1.0000x reference)
import jax
import jax.numpy as jnp
from jax.experimental import pallas as pl
from jax.experimental.pallas import tpu as pltpu


_IMGS = 384
_NCH = 6
_CH = _IMGS // _NCH
_SLOTS = 3


def _copy_kernel(x_hbm, o_hbm, buf, in_sems, out_sems):
    def cin(i, slot):
        return pltpu.make_async_copy(
            x_hbm.at[pl.ds(i * _CH, _CH)], buf.at[slot], in_sems.at[slot])

    def cout(i, slot):
        return pltpu.make_async_copy(
            buf.at[slot], o_hbm.at[pl.ds(i * _CH, _CH)], out_sems.at[slot])

    for k in range(_SLOTS):
        cin(k, k).start()
    pending = None
    for i in range(_NCH):
        s = i % _SLOTS
        if pending is not None:
            pj, ps = pending
            cout(pj, ps).wait()
            cin(pj + _SLOTS, ps).start()
            pending = None
        cin(i, s).wait()
        cout(i, s).start()
        if i + _SLOTS < _NCH:
            pending = (i, s)
    for i in range(_NCH - _SLOTS, _NCH):
        cout(i, i % _SLOTS).wait()


def kernel(feature_batch, box_batch):
    x = feature_batch.reshape(_IMGS, 224, 224)
    out = pl.pallas_call(
        _copy_kernel,
        in_specs=[pl.BlockSpec(memory_space=pltpu.MemorySpace.HBM)],
        out_specs=pl.BlockSpec(memory_space=pltpu.MemorySpace.HBM),
        out_shape=jax.ShapeDtypeStruct((_IMGS, 224, 224), jnp.float32),
        scratch_shapes=[
            pltpu.VMEM((_SLOTS, _CH, 224, 224), jnp.float32),
            pltpu.SemaphoreType.DMA((_SLOTS,)),
            pltpu.SemaphoreType.DMA((_SLOTS,)),
        ],
    )(x)
    return out.reshape(feature_batch.shape)
